# Initial kernel scaffold; baseline (speedup 1.0000x reference)
import jax
import jax.numpy as jnp
from jax.experimental import pallas as pl


def kernel(x, gate_w, w1, w3, w2):
    raise NotImplementedError("write your pallas kernel here")


# speedup vs baseline: 22.4621x; 22.4621x over previous
import functools

import jax
import jax.numpy as jnp
from jax import lax
from jax.experimental import pallas as pl
from jax.experimental.pallas import tpu as pltpu
from jax.experimental.pallas import tpu_sc as plsc

DIM = 768
HID = 1024
E = 64
K = 2
S = 2048
N = S * K
TILE = 128
NT = 96
CAP = NT * TILE

NC = 2
NS = 16
NW = NC * NS
ROWS_W = N // NW
TOK_W = S // NW

_f32 = jnp.float32
_i32 = jnp.int32


def _gate_route_body(tok_ref, gw_ref, pos_ref, te_ref, ntl_ref, wbig_ref,
                     ef_ref, rank_ref):
    x = tok_ref[...]
    gw = gw_ref[...]
    logits = lax.dot_general(x, gw, (((1,), (1,)), ((), ())),
                             preferred_element_type=_f32)

    iota_e = lax.broadcasted_iota(_i32, (S, E), 1)
    m1 = jnp.max(logits, axis=1, keepdims=True)
    i1 = jnp.min(jnp.where(logits == m1, iota_e, E), axis=1, keepdims=True)
    masked = jnp.where(iota_e == i1, _f32(-1e30), logits)
    m2 = jnp.max(masked, axis=1, keepdims=True)
    i2 = jnp.min(jnp.where(masked == m2, iota_e, E), axis=1, keepdims=True)

    e2 = jnp.exp(m2 - m1)
    den = 1.0 + e2
    wA = 1.0 / den
    wB = e2 / den
    wbig_ref[pl.ds(0, S), :] = jnp.broadcast_to(wA, (S, DIM))
    wbig_ref[pl.ds(S, S), :] = jnp.broadcast_to(wB, (S, DIM))

    ef_ref[pl.ds(0, S), :] = i1
    ef_ref[pl.ds(S, S), :] = i2

    nblk = N // TILE
    iota_be = lax.broadcasted_iota(_i32, (TILE, E), 1)
    r0 = lax.broadcasted_iota(_i32, (TILE, TILE), 0)
    c0 = lax.broadcasted_iota(_i32, (TILE, TILE), 1)
    tri = (r0 >= c0).astype(_f32)

    def blk1(b, counts):
        eb = ef_ref[pl.ds(b * TILE, TILE), :]
        oh = (eb == iota_be).astype(_f32)
        cum = lax.dot_general(tri, oh, (((1,), (0,)), ((), ())),
                              preferred_element_type=_f32)
        rank = jnp.sum((cum + counts) * oh, axis=1, keepdims=True) - 1.0
        rank_ref[pl.ds(b * TILE, TILE), :] = rank
        return counts + jnp.sum(oh, axis=0, keepdims=True)

    counts = lax.fori_loop(0, nblk, blk1, jnp.zeros((1, E), _f32))

    ntile = jnp.floor((counts + _f32(TILE - 1)) * _f32(1.0 / TILE))
    e_r = lax.broadcasted_iota(_i32, (E, E), 0)
    e_c = lax.broadcasted_iota(_i32, (E, E), 1)
    excl = (e_r < e_c).astype(_f32)
    off_t = lax.dot_general(ntile, excl, (((1,), (0,)), ((), ())),
                            preferred_element_type=_f32)
    off_r = off_t * _f32(TILE)
    ntl_ref[...] = jnp.sum(ntile, axis=1, keepdims=True).astype(_i32)

    def blk2(b, carry):
        eb = ef_ref[pl.ds(b * TILE, TILE), :]
        oh = (eb == iota_be).astype(_f32)
        offsel = jnp.sum(oh * off_r, axis=1, keepdims=True)
        pos = rank_ref[pl.ds(b * TILE, TILE), :] + offsel
        pos_ref[pl.ds(b * TILE, TILE), :] = pos.astype(_i32)
        return carry

    lax.fori_loop(0, nblk, blk2, 0)

    t_iota = lax.broadcasted_iota(_i32, (TILE, E), 0).astype(_f32)
    te = jnp.sum((off_t <= t_iota).astype(_f32), axis=1, keepdims=True) - 1.0
    te_ref[...] = te.astype(_i32)


def _gate_route(tokens, gate_w):
    return pl.pallas_call(
        _gate_route_body,
        out_shape=(
            jax.ShapeDtypeStruct((N, 1), _i32),
            jax.ShapeDtypeStruct((TILE, 1), _i32),
            jax.ShapeDtypeStruct((1, 1), _i32),
            jax.ShapeDtypeStruct((N, DIM), _f32),
        ),
        scratch_shapes=[
            pltpu.VMEM((N, 1), _i32),
            pltpu.VMEM((N, 1), _f32),
        ],
    )(tokens, gate_w)


_HTOK = TOK_W // 2
_CHUNKS = DIM // 16


@functools.lru_cache(maxsize=1)
def _sc_kernels():
    mesh = plsc.VectorSubcoreMesh(core_axis_name="c", subcore_axis_name="s")

    @functools.partial(
        pl.kernel,
        out_type=jax.ShapeDtypeStruct((CAP, DIM), _f32),
        mesh=mesh,
        scratch_types=[
            pltpu.VMEM((64,), _i32),
            pltpu.VMEM((64,), _i32),
            pltpu.VMEM((64, DIM), _f32),
            pltpu.SemaphoreType.DMA,
        ],
    )
    def _dispatch_sc(tok_hbm, pos_hbm, xs_hbm, src_v, pos_v, rows_v, sem):
        wid = lax.axis_index("s") * NC + lax.axis_index("c")
        base = wid * ROWS_W
        for h in range(ROWS_W // 64):
            b = base + h * 64
            for c in range(4):
                src_v[pl.ds(c * 16, 16)] = (lax.iota(_i32, 16) + (b + c * 16)) & (S - 1)
            pltpu.sync_copy(pos_hbm.at[pl.ds(b, 64)], pos_v)
            pltpu.async_copy(tok_hbm.at[src_v], rows_v, sem).wait()
            pltpu.sync_copy(rows_v, xs_hbm.at[pos_v])

    @functools.partial(
        pl.kernel,
        out_type=jax.ShapeDtypeStruct((S, DIM), _f32),
        mesh=mesh,
        scratch_types=[
            pltpu.VMEM((_HTOK,), _i32),
            pltpu.VMEM((_HTOK, DIM), _f32),
            pltpu.VMEM((_HTOK, DIM), _f32),
            pltpu.VMEM((_HTOK, DIM), _f32),
            pltpu.SemaphoreType.DMA,
        ],
    )
    def _combine_sc(os_hbm, pos_hbm, wbig_hbm, y_hbm, pos_v, row_v, w_v, acc_v, sem):
        wid = lax.axis_index("s") * NC + lax.axis_index("c")
        tbase = wid * TOK_W
        for h in range(TOK_W // _HTOK):
            tb = tbase + h * _HTOK
            for k in range(K):
                pltpu.sync_copy(pos_hbm.at[pl.ds(k * S + tb, _HTOK)], pos_v)
                pltpu.async_copy(os_hbm.at[pos_v], row_v, sem).wait()
                pltpu.sync_copy(wbig_hbm.at[pl.ds(k * S + tb, _HTOK)], w_v)
                for r in range(_HTOK):
                    def chunk(c, carry, r=r, k=k):
                        sl = pl.ds(c * 16, 16)
                        prod = row_v[r, sl] * w_v[r, sl]
                        if k == 0:
                            acc_v[r, sl] = prod
                        else:
                            acc_v[r, sl] = acc_v[r, sl] + prod
                        return carry
                    lax.fori_loop(0, _CHUNKS, chunk, 0)
            pltpu.sync_copy(acc_v, y_hbm.at[pl.ds(tb, _HTOK)])

    return _dispatch_sc, _combine_sc


def _ffn_body(te_s, ntl_s, x_ref, w1_ref, w3_ref, w2_ref, o_ref):
    @pl.when(pl.program_id(0) < ntl_s[0])
    def _():
        x = x_ref[...]
        g = lax.dot_general(x, w1_ref[0], (((1,), (1,)), ((), ())),
                            preferred_element_type=_f32)
        g = g * jax.nn.sigmoid(g)
        u = lax.dot_general(x, w3_ref[0], (((1,), (1,)), ((), ())),
                            preferred_element_type=_f32)
        o_ref[...] = lax.dot_general(g * u, w2_ref[0], (((1,), (1,)), ((), ())),
                                     preferred_element_type=_f32)


def _ffn(te, ntl, x_sorted, w1, w3, w2):
    grid_spec = pltpu.PrefetchScalarGridSpec(
        num_scalar_prefetch=2,
        grid=(NT,),
        in_specs=[
            pl.BlockSpec((TILE, DIM), lambda i, te_s, ntl_s: (i, 0)),
            pl.BlockSpec((1, HID, DIM), lambda i, te_s, ntl_s: (te_s[i], 0, 0)),
            pl.BlockSpec((1, HID, DIM), lambda i, te_s, ntl_s: (te_s[i], 0, 0)),
            pl.BlockSpec((1, DIM, HID), lambda i, te_s, ntl_s: (te_s[i], 0, 0)),
        ],
        out_specs=pl.BlockSpec((TILE, DIM), lambda i, te_s, ntl_s: (i, 0)),
    )
    return pl.pallas_call(
        _ffn_body,
        grid_spec=grid_spec,
        out_shape=jax.ShapeDtypeStruct((CAP, DIM), _f32),
    )(te, ntl, x_sorted, w1, w3, w2)


@jax.jit
def kernel(x, gate_w, w1, w3, w2):
    dispatch_sc, combine_sc = _sc_kernels()
    b, s, d = x.shape
    tokens = x.reshape(S, DIM)
    pos2, te2, ntl2, wbig = _gate_route(tokens, gate_w)
    pos = pos2.reshape(N)
    te = te2.reshape(TILE)
    ntl = ntl2.reshape(1)
    x_sorted = dispatch_sc(tokens, pos)
    out_sorted = _ffn(te, ntl, x_sorted, w1, w3, w2)
    y = combine_sc(out_sorted, pos, wbig)
    return y.reshape(b, s, d)


# speedup vs baseline: 22.4703x; 1.0004x over previous
import functools

import jax
import jax.numpy as jnp
from jax import lax
from jax.experimental import pallas as pl
from jax.experimental.pallas import tpu as pltpu
from jax.experimental.pallas import tpu_sc as plsc

DIM = 768
HID = 1024
E = 64
K = 2
S = 2048
N = S * K
TILE = 128
NT = 96
CAP = NT * TILE

NC = 2
NS = 16
NW = NC * NS
ROWS_W = N // NW
TOK_W = S // NW

_f32 = jnp.float32
_i32 = jnp.int32


def _gate_route_body(tok_ref, gw_ref, pos_ref, te_ref, ntl_ref, wbig_ref,
                     ef_ref, rank_ref):
    x = tok_ref[...]
    gw = gw_ref[...]
    logits = lax.dot_general(x, gw, (((1,), (1,)), ((), ())),
                             preferred_element_type=_f32)

    iota_e = lax.broadcasted_iota(_i32, (S, E), 1)
    m1 = jnp.max(logits, axis=1, keepdims=True)
    i1 = jnp.min(jnp.where(logits == m1, iota_e, E), axis=1, keepdims=True)
    masked = jnp.where(iota_e == i1, _f32(-1e30), logits)
    m2 = jnp.max(masked, axis=1, keepdims=True)
    i2 = jnp.min(jnp.where(masked == m2, iota_e, E), axis=1, keepdims=True)

    e2 = jnp.exp(m2 - m1)
    den = 1.0 + e2
    wA = 1.0 / den
    wB = e2 / den
    wbig_ref[pl.ds(0, S), :] = jnp.broadcast_to(wA, (S, DIM))
    wbig_ref[pl.ds(S, S), :] = jnp.broadcast_to(wB, (S, DIM))

    ef_ref[pl.ds(0, S), :] = i1
    ef_ref[pl.ds(S, S), :] = i2

    nblk = N // TILE
    iota_be = lax.broadcasted_iota(_i32, (TILE, E), 1)
    r0 = lax.broadcasted_iota(_i32, (TILE, TILE), 0)
    c0 = lax.broadcasted_iota(_i32, (TILE, TILE), 1)
    tri = (r0 >= c0).astype(_f32)

    def blk1(b, counts):
        eb = ef_ref[pl.ds(b * TILE, TILE), :]
        oh = (eb == iota_be).astype(_f32)
        cum = lax.dot_general(tri, oh, (((1,), (0,)), ((), ())),
                              preferred_element_type=_f32)
        rank = jnp.sum((cum + counts) * oh, axis=1, keepdims=True) - 1.0
        rank_ref[pl.ds(b * TILE, TILE), :] = rank
        return counts + jnp.sum(oh, axis=0, keepdims=True)

    counts = lax.fori_loop(0, nblk, blk1, jnp.zeros((1, E), _f32))

    ntile = jnp.floor((counts + _f32(TILE - 1)) * _f32(1.0 / TILE))
    e_r = lax.broadcasted_iota(_i32, (E, E), 0)
    e_c = lax.broadcasted_iota(_i32, (E, E), 1)
    excl = (e_r < e_c).astype(_f32)
    off_t = lax.dot_general(ntile, excl, (((1,), (0,)), ((), ())),
                            preferred_element_type=_f32)
    off_r = off_t * _f32(TILE)
    ntl_ref[...] = jnp.sum(ntile, axis=1, keepdims=True).astype(_i32)

    def blk2(b, carry):
        eb = ef_ref[pl.ds(b * TILE, TILE), :]
        oh = (eb == iota_be).astype(_f32)
        offsel = jnp.sum(oh * off_r, axis=1, keepdims=True)
        pos = rank_ref[pl.ds(b * TILE, TILE), :] + offsel
        pos_ref[pl.ds(b * TILE, TILE), :] = pos.astype(_i32)
        return carry

    lax.fori_loop(0, nblk, blk2, 0)

    t_iota = lax.broadcasted_iota(_i32, (TILE, E), 0).astype(_f32)
    te = jnp.sum((off_t <= t_iota).astype(_f32), axis=1, keepdims=True) - 1.0
    te_ref[...] = te.astype(_i32)


def _gate_route(tokens, gate_w):
    return pl.pallas_call(
        _gate_route_body,
        out_shape=(
            jax.ShapeDtypeStruct((N, 1), _i32),
            jax.ShapeDtypeStruct((TILE, 1), _i32),
            jax.ShapeDtypeStruct((1, 1), _i32),
            jax.ShapeDtypeStruct((N, DIM), _f32),
        ),
        scratch_shapes=[
            pltpu.VMEM((N, 1), _i32),
            pltpu.VMEM((N, 1), _f32),
        ],
    )(tokens, gate_w)


_HTOK = TOK_W // 2
_CHUNKS = DIM // 16


@functools.lru_cache(maxsize=1)
def _sc_kernels():
    mesh = plsc.VectorSubcoreMesh(core_axis_name="c", subcore_axis_name="s")

    @functools.partial(
        pl.kernel,
        out_type=jax.ShapeDtypeStruct((CAP, DIM), _f32),
        mesh=mesh,
        scratch_types=[
            pltpu.VMEM((64,), _i32),
            pltpu.VMEM((64,), _i32),
            pltpu.VMEM((64, DIM), _f32),
            pltpu.SemaphoreType.DMA,
        ],
    )
    def _dispatch_sc(tok_hbm, pos_hbm, xs_hbm, src_v, pos_v, rows_v, sem):
        wid = lax.axis_index("s") * NC + lax.axis_index("c")
        base = wid * ROWS_W
        for h in range(ROWS_W // 64):
            b = base + h * 64
            for c in range(4):
                src_v[pl.ds(c * 16, 16)] = (lax.iota(_i32, 16) + (b + c * 16)) & (S - 1)
            pltpu.sync_copy(pos_hbm.at[pl.ds(b, 64)], pos_v)
            pltpu.async_copy(tok_hbm.at[src_v], rows_v, sem).wait()
            pltpu.sync_copy(rows_v, xs_hbm.at[pos_v])

    @functools.partial(
        pl.kernel,
        out_type=jax.ShapeDtypeStruct((S, DIM), _f32),
        mesh=mesh,
        scratch_types=[
            pltpu.VMEM((_HTOK,), _i32),
            pltpu.VMEM((_HTOK, DIM), _f32),
            pltpu.VMEM((_HTOK, DIM), _f32),
            pltpu.VMEM((_HTOK, DIM), _f32),
            pltpu.SemaphoreType.DMA,
        ],
    )
    def _combine_sc(os_hbm, pos_hbm, wbig_hbm, y_hbm, pos_v, row_v, w_v, acc_v, sem):
        wid = lax.axis_index("s") * NC + lax.axis_index("c")
        tbase = wid * TOK_W
        for h in range(TOK_W // _HTOK):
            tb = tbase + h * _HTOK
            for k in range(K):
                pltpu.sync_copy(pos_hbm.at[pl.ds(k * S + tb, _HTOK)], pos_v)
                pltpu.async_copy(os_hbm.at[pos_v], row_v, sem).wait()
                pltpu.sync_copy(wbig_hbm.at[pl.ds(k * S + tb, _HTOK)], w_v)
                for r in range(_HTOK):
                    def chunk(c, carry, r=r, k=k):
                        sl = pl.ds(c * 16, 16)
                        prod = row_v[r, sl] * w_v[r, sl]
                        if k == 0:
                            acc_v[r, sl] = prod
                        else:
                            acc_v[r, sl] = acc_v[r, sl] + prod
                        return carry
                    lax.fori_loop(0, _CHUNKS, chunk, 0)
            pltpu.sync_copy(acc_v, y_hbm.at[pl.ds(tb, _HTOK)])

    return _dispatch_sc, _combine_sc


def _ffn_body(te_s, ntl_s, x_ref, w1_ref, w3_ref, w2_ref, o_ref):
    @pl.when(pl.program_id(0) < ntl_s[0])
    def _():
        bf16 = jnp.bfloat16
        x = x_ref[...].astype(bf16)
        g = lax.dot_general(x, w1_ref[0].astype(bf16), (((1,), (1,)), ((), ())),
                            preferred_element_type=_f32)
        g = g * jax.nn.sigmoid(g)
        u = lax.dot_general(x, w3_ref[0].astype(bf16), (((1,), (1,)), ((), ())),
                            preferred_element_type=_f32)
        h = (g * u).astype(bf16)
        o_ref[...] = lax.dot_general(h, w2_ref[0].astype(bf16), (((1,), (1,)), ((), ())),
                                     preferred_element_type=_f32)


def _ffn(te, ntl, x_sorted, w1, w3, w2):
    grid_spec = pltpu.PrefetchScalarGridSpec(
        num_scalar_prefetch=2,
        grid=(NT,),
        in_specs=[
            pl.BlockSpec((TILE, DIM), lambda i, te_s, ntl_s: (i, 0)),
            pl.BlockSpec((1, HID, DIM), lambda i, te_s, ntl_s: (te_s[i], 0, 0)),
            pl.BlockSpec((1, HID, DIM), lambda i, te_s, ntl_s: (te_s[i], 0, 0)),
            pl.BlockSpec((1, DIM, HID), lambda i, te_s, ntl_s: (te_s[i], 0, 0)),
        ],
        out_specs=pl.BlockSpec((TILE, DIM), lambda i, te_s, ntl_s: (i, 0)),
    )
    return pl.pallas_call(
        _ffn_body,
        grid_spec=grid_spec,
        out_shape=jax.ShapeDtypeStruct((CAP, DIM), _f32),
    )(te, ntl, x_sorted, w1, w3, w2)


@jax.jit
def kernel(x, gate_w, w1, w3, w2):
    dispatch_sc, combine_sc = _sc_kernels()
    b, s, d = x.shape
    tokens = x.reshape(S, DIM)
    pos2, te2, ntl2, wbig = _gate_route(tokens, gate_w)
    pos = pos2.reshape(N)
    te = te2.reshape(TILE)
    ntl = ntl2.reshape(1)
    x_sorted = dispatch_sc(tokens, pos)
    out_sorted = _ffn(te, ntl, x_sorted, w1, w3, w2)
    y = combine_sc(out_sorted, pos, wbig)
    return y.reshape(b, s, d)


# speedup vs baseline: 25.2863x; 1.1253x over previous
import functools

import jax
import jax.numpy as jnp
from jax import lax
from jax.experimental import pallas as pl
from jax.experimental.pallas import tpu as pltpu
from jax.experimental.pallas import tpu_sc as plsc

DIM = 768
HID = 1024
E = 64
K = 2
S = 2048
N = S * K
TILE = 128
NT = 96
CAP = NT * TILE

NC = 2
NS = 16
NW = NC * NS
ROWS_W = N // NW
TOK_W = S // NW

_f32 = jnp.float32
_i32 = jnp.int32


_RB = 1024


def _gate_route_body(tok_ref, gw_ref, pos_ref, te_ref, ntl_ref, wflat_ref,
                     oh_ref, rank_ref):
    x = tok_ref[...]
    gw = gw_ref[...]
    logits = lax.dot_general(x, gw, (((1,), (1,)), ((), ())),
                             preferred_element_type=_f32)

    iota_e = lax.broadcasted_iota(_i32, (S, E), 1)
    m1 = jnp.max(logits, axis=1, keepdims=True)
    i1 = jnp.min(jnp.where(logits == m1, iota_e, E), axis=1, keepdims=True)
    masked = jnp.where(iota_e == i1, _f32(-1e30), logits)
    m2 = jnp.max(masked, axis=1, keepdims=True)
    i2 = jnp.min(jnp.where(masked == m2, iota_e, E), axis=1, keepdims=True)

    e2 = jnp.exp(m2 - m1)
    den = 1.0 + e2
    wflat_ref[pl.ds(0, S), :] = jnp.broadcast_to(1.0 / den, (S, 16))
    wflat_ref[pl.ds(S, S), :] = jnp.broadcast_to(e2 / den, (S, 16))

    iota_se = lax.broadcasted_iota(_i32, (S, E), 1)
    oh_ref[pl.ds(0, S), :] = (i1 == iota_se).astype(_f32)
    oh_ref[pl.ds(S, S), :] = (i2 == iota_se).astype(_f32)

    nblk = N // _RB
    r0 = lax.broadcasted_iota(_i32, (_RB, _RB), 0)
    c0 = lax.broadcasted_iota(_i32, (_RB, _RB), 1)
    tri = (r0 >= c0).astype(_f32)

    def blk1(b, counts):
        oh = oh_ref[pl.ds(b * _RB, _RB), :]
        cum = lax.dot_general(tri, oh, (((1,), (0,)), ((), ())),
                              preferred_element_type=_f32)
        rank = jnp.sum((cum + counts) * oh, axis=1, keepdims=True) - 1.0
        rank_ref[pl.ds(b * _RB, _RB), :] = rank
        return counts + jnp.sum(oh, axis=0, keepdims=True)

    counts = lax.fori_loop(0, nblk, blk1, jnp.zeros((1, E), _f32))

    ntile = jnp.floor((counts + _f32(TILE - 1)) * _f32(1.0 / TILE))
    e_r = lax.broadcasted_iota(_i32, (E, E), 0)
    e_c = lax.broadcasted_iota(_i32, (E, E), 1)
    excl = (e_r < e_c).astype(_f32)
    off_t = lax.dot_general(ntile, excl, (((1,), (0,)), ((), ())),
                            preferred_element_type=_f32)
    off_r = off_t * _f32(TILE)
    ntl_ref[...] = jnp.sum(ntile, axis=1, keepdims=True).astype(_i32)

    offsel = jnp.sum(oh_ref[...] * off_r, axis=1, keepdims=True)
    pos_ref[...] = (rank_ref[...] + offsel).astype(_i32)

    t_iota = lax.broadcasted_iota(_i32, (TILE, E), 0).astype(_f32)
    te = jnp.sum((off_t <= t_iota).astype(_f32), axis=1, keepdims=True) - 1.0
    te_ref[...] = te.astype(_i32)


def _gate_route(tokens, gate_w):
    return pl.pallas_call(
        _gate_route_body,
        out_shape=(
            jax.ShapeDtypeStruct((N, 1), _i32),
            jax.ShapeDtypeStruct((TILE, 1), _i32),
            jax.ShapeDtypeStruct((1, 1), _i32),
            jax.ShapeDtypeStruct((N, 16), _f32),
        ),
        scratch_shapes=[
            pltpu.VMEM((N, E), _f32),
            pltpu.VMEM((N, 1), _f32),
        ],
    )(tokens, gate_w)


_HTOK = TOK_W // 2
_CHUNKS = DIM // 16


@functools.lru_cache(maxsize=1)
def _sc_kernels():
    mesh = plsc.VectorSubcoreMesh(core_axis_name="c", subcore_axis_name="s")

    @functools.partial(
        pl.kernel,
        out_type=jax.ShapeDtypeStruct((CAP, DIM), _f32),
        mesh=mesh,
        scratch_types=[
            pltpu.VMEM((64,), _i32),
            pltpu.VMEM((64,), _i32),
            pltpu.VMEM((64, DIM), _f32),
            pltpu.SemaphoreType.DMA,
        ],
    )
    def _dispatch_sc(tok_hbm, pos_hbm, xs_hbm, src_v, pos_v, rows_v, sem):
        wid = lax.axis_index("s") * NC + lax.axis_index("c")
        base = wid * ROWS_W
        for h in range(ROWS_W // 64):
            b = base + h * 64
            for c in range(4):
                src_v[pl.ds(c * 16, 16)] = (lax.iota(_i32, 16) + (b + c * 16)) & (S - 1)
            pltpu.sync_copy(pos_hbm.at[pl.ds(b, 64)], pos_v)
            pltpu.async_copy(tok_hbm.at[src_v], rows_v, sem).wait()
            pltpu.sync_copy(rows_v, xs_hbm.at[pos_v])

    @functools.partial(
        pl.kernel,
        out_type=jax.ShapeDtypeStruct((S, DIM), _f32),
        mesh=mesh,
        scratch_types=[
            pltpu.VMEM((_HTOK,), _i32),
            pltpu.VMEM((_HTOK * 16,), _f32),
            pltpu.VMEM((_HTOK * 16,), _f32),
            pltpu.VMEM((_HTOK, DIM), _f32),
            pltpu.VMEM((_HTOK, DIM), _f32),
            pltpu.SemaphoreType.DMA,
        ],
    )
    def _combine_sc(os_hbm, pos_hbm, wf_hbm, y_hbm,
                    pos_v, wa_v, wb_v, row_v, acc_v, sem):
        wid = lax.axis_index("s") * NC + lax.axis_index("c")
        tbase = wid * TOK_W
        for h in range(TOK_W // _HTOK):
            tb = tbase + h * _HTOK
            pltpu.sync_copy(pos_hbm.at[pl.ds(tb, _HTOK)], pos_v)
            pltpu.async_copy(os_hbm.at[pos_v], acc_v, sem).wait()
            pltpu.sync_copy(pos_hbm.at[pl.ds(S + tb, _HTOK)], pos_v)
            pltpu.async_copy(os_hbm.at[pos_v], row_v, sem).wait()
            pltpu.sync_copy(wf_hbm.at[pl.ds(tb * 16, _HTOK * 16)], wa_v)
            pltpu.sync_copy(wf_hbm.at[pl.ds((S + tb) * 16, _HTOK * 16)], wb_v)
            for r in range(_HTOK):
                wa = wa_v[pl.ds(r * 16, 16)]
                wb = wb_v[pl.ds(r * 16, 16)]

                def chunk(c, carry, r=r, wa=wa, wb=wb):
                    for u in range(4):
                        sl = pl.ds(c * 64 + u * 16, 16)
                        acc_v[r, sl] = acc_v[r, sl] * wa + row_v[r, sl] * wb
                    return carry
                lax.fori_loop(0, _CHUNKS // 4, chunk, 0)
            pltpu.sync_copy(acc_v, y_hbm.at[pl.ds(tb, _HTOK)])

    return _dispatch_sc, _combine_sc


def _ffn_body(te_s, ntl_s, x_ref, w1_ref, w3_ref, w2_ref, o_ref):
    @pl.when(pl.program_id(0) < ntl_s[0])
    def _():
        bf16 = jnp.bfloat16
        x = x_ref[...].astype(bf16)
        g = lax.dot_general(x, w1_ref[0].astype(bf16), (((1,), (1,)), ((), ())),
                            preferred_element_type=_f32)
        g = g * jax.nn.sigmoid(g)
        u = lax.dot_general(x, w3_ref[0].astype(bf16), (((1,), (1,)), ((), ())),
                            preferred_element_type=_f32)
        h = (g * u).astype(bf16)
        o_ref[...] = lax.dot_general(h, w2_ref[0].astype(bf16), (((1,), (1,)), ((), ())),
                                     preferred_element_type=_f32)


def _ffn(te, ntl, x_sorted, w1, w3, w2):
    grid_spec = pltpu.PrefetchScalarGridSpec(
        num_scalar_prefetch=2,
        grid=(NT,),
        in_specs=[
            pl.BlockSpec((TILE, DIM), lambda i, te_s, ntl_s: (i, 0)),
            pl.BlockSpec((1, HID, DIM), lambda i, te_s, ntl_s: (te_s[i], 0, 0)),
            pl.BlockSpec((1, HID, DIM), lambda i, te_s, ntl_s: (te_s[i], 0, 0)),
            pl.BlockSpec((1, DIM, HID), lambda i, te_s, ntl_s: (te_s[i], 0, 0)),
        ],
        out_specs=pl.BlockSpec((TILE, DIM), lambda i, te_s, ntl_s: (i, 0)),
    )
    return pl.pallas_call(
        _ffn_body,
        grid_spec=grid_spec,
        out_shape=jax.ShapeDtypeStruct((CAP, DIM), _f32),
    )(te, ntl, x_sorted, w1, w3, w2)


@jax.jit
def kernel(x, gate_w, w1, w3, w2):
    dispatch_sc, combine_sc = _sc_kernels()
    b, s, d = x.shape
    tokens = x.reshape(S, DIM)
    pos2, te2, ntl2, wflat2 = _gate_route(tokens, gate_w)
    pos = pos2.reshape(N)
    te = te2.reshape(TILE)
    ntl = ntl2.reshape(1)
    wflat = wflat2.reshape(N * 16)
    x_sorted = dispatch_sc(tokens, pos)
    out_sorted = _ffn(te, ntl, x_sorted, w1, w3, w2)
    y = combine_sc(out_sorted, pos, wflat)
    return y.reshape(b, s, d)


# speedup vs baseline: 26.5069x; 1.0483x over previous
import functools

import jax
import jax.numpy as jnp
from jax import lax
from jax.experimental import pallas as pl
from jax.experimental.pallas import tpu as pltpu
from jax.experimental.pallas import tpu_sc as plsc

DIM = 768
HID = 1024
E = 64
K = 2
S = 2048
N = S * K
TILE = 128
NT = 96
CAP = NT * TILE

NC = 2
NS = 16
NW = NC * NS
ROWS_W = N // NW
TOK_W = S // NW

_f32 = jnp.float32
_i32 = jnp.int32


_RB = 1024


def _gate_route_body(tok_ref, gw_ref, pos_ref, te_ref, ntl_ref, wflat_ref,
                     oh_ref, rank_ref):
    x = tok_ref[...]
    gw = gw_ref[...]
    logits = lax.dot_general(x, gw, (((1,), (1,)), ((), ())),
                             preferred_element_type=_f32)

    iota_e = lax.broadcasted_iota(_i32, (S, E), 1)
    m1 = jnp.max(logits, axis=1, keepdims=True)
    i1 = jnp.min(jnp.where(logits == m1, iota_e, E), axis=1, keepdims=True)
    masked = jnp.where(iota_e == i1, _f32(-1e30), logits)
    m2 = jnp.max(masked, axis=1, keepdims=True)
    i2 = jnp.min(jnp.where(masked == m2, iota_e, E), axis=1, keepdims=True)

    e2 = jnp.exp(m2 - m1)
    den = 1.0 + e2
    wflat_ref[pl.ds(0, S), :] = jnp.broadcast_to(1.0 / den, (S, 16))
    wflat_ref[pl.ds(S, S), :] = jnp.broadcast_to(e2 / den, (S, 16))

    iota_se = lax.broadcasted_iota(_i32, (S, E), 1)
    oh_ref[pl.ds(0, S), :] = (i1 == iota_se).astype(_f32)
    oh_ref[pl.ds(S, S), :] = (i2 == iota_se).astype(_f32)

    nblk = N // _RB
    r0 = lax.broadcasted_iota(_i32, (_RB, _RB), 0)
    c0 = lax.broadcasted_iota(_i32, (_RB, _RB), 1)
    tri = (r0 >= c0).astype(_f32)

    def blk1(b, counts):
        oh = oh_ref[pl.ds(b * _RB, _RB), :]
        cum = lax.dot_general(tri, oh, (((1,), (0,)), ((), ())),
                              preferred_element_type=_f32)
        rank = jnp.sum((cum + counts) * oh, axis=1, keepdims=True) - 1.0
        rank_ref[pl.ds(b * _RB, _RB), :] = rank
        return counts + jnp.sum(oh, axis=0, keepdims=True)

    counts = lax.fori_loop(0, nblk, blk1, jnp.zeros((1, E), _f32))

    ntile = jnp.floor((counts + _f32(TILE - 1)) * _f32(1.0 / TILE))
    e_r = lax.broadcasted_iota(_i32, (E, E), 0)
    e_c = lax.broadcasted_iota(_i32, (E, E), 1)
    excl = (e_r < e_c).astype(_f32)
    off_t = lax.dot_general(ntile, excl, (((1,), (0,)), ((), ())),
                            preferred_element_type=_f32)
    off_r = off_t * _f32(TILE)
    ntl_ref[...] = jnp.sum(ntile, axis=1, keepdims=True).astype(_i32)

    offsel = jnp.sum(oh_ref[...] * off_r, axis=1, keepdims=True)
    pos_ref[...] = (rank_ref[...] + offsel).astype(_i32)

    t_iota = lax.broadcasted_iota(_i32, (TILE, E), 0).astype(_f32)
    te = jnp.sum((off_t <= t_iota).astype(_f32), axis=1, keepdims=True) - 1.0
    te_ref[...] = te.astype(_i32)


def _gate_route(tokens, gate_w):
    return pl.pallas_call(
        _gate_route_body,
        out_shape=(
            jax.ShapeDtypeStruct((N, 1), _i32),
            jax.ShapeDtypeStruct((TILE, 1), _i32),
            jax.ShapeDtypeStruct((1, 1), _i32),
            jax.ShapeDtypeStruct((N, 16), _f32),
        ),
        scratch_shapes=[
            pltpu.VMEM((N, E), _f32),
            pltpu.VMEM((N, 1), _f32),
        ],
    )(tokens, gate_w)


_HTOK = TOK_W // 2
_CHUNKS = DIM // 16


@functools.lru_cache(maxsize=1)
def _sc_kernels():
    mesh = plsc.VectorSubcoreMesh(core_axis_name="c", subcore_axis_name="s")

    @functools.partial(
        pl.kernel,
        out_type=jax.ShapeDtypeStruct((CAP, DIM), _f32),
        mesh=mesh,
        scratch_types=[
            pltpu.VMEM((64,), _i32),
            pltpu.VMEM((64,), _i32),
            pltpu.VMEM((64,), _i32),
            pltpu.VMEM((64,), _i32),
            pltpu.VMEM((64, DIM), _f32),
            pltpu.VMEM((64, DIM), _f32),
            pltpu.SemaphoreType.DMA,
            pltpu.SemaphoreType.DMA,
        ],
    )
    def _dispatch_sc(tok_hbm, pos_hbm, xs_hbm,
                     src0, src1, pos0, pos1, rows0, rows1, gsem, ssem):
        wid = lax.axis_index("s") * NC + lax.axis_index("c")
        base = wid * ROWS_W
        srcs, poss, rows = [src0, src1], [pos0, pos1], [rows0, rows1]
        gd = []
        for h in range(2):
            b = base + h * 64
            for c in range(4):
                srcs[h][pl.ds(c * 16, 16)] = (
                    lax.iota(_i32, 16) + (b + c * 16)) & (S - 1)
            pltpu.sync_copy(pos_hbm.at[pl.ds(b, 64)], poss[h])
            gd.append(pltpu.async_copy(tok_hbm.at[srcs[h]], rows[h], gsem))
        sd = []
        for h in range(2):
            gd[h].wait()
            sd.append(pltpu.async_copy(rows[h], xs_hbm.at[poss[h]], ssem))
        for d in sd:
            d.wait()

    @functools.partial(
        pl.kernel,
        out_type=jax.ShapeDtypeStruct((S, DIM), _f32),
        mesh=mesh,
        scratch_types=[
            pltpu.VMEM((_HTOK,), _i32),
            pltpu.VMEM((_HTOK,), _i32),
            pltpu.VMEM((_HTOK,), _i32),
            pltpu.VMEM((_HTOK,), _i32),
            pltpu.VMEM((_HTOK * 16,), _f32),
            pltpu.VMEM((_HTOK * 16,), _f32),
            pltpu.VMEM((_HTOK * 16,), _f32),
            pltpu.VMEM((_HTOK * 16,), _f32),
            pltpu.VMEM((_HTOK, DIM), _f32),
            pltpu.VMEM((_HTOK, DIM), _f32),
            pltpu.VMEM((_HTOK, DIM), _f32),
            pltpu.VMEM((_HTOK, DIM), _f32),
            pltpu.SemaphoreType.DMA,
        ],
    )
    def _combine_sc(os_hbm, pos_hbm, wf_hbm, y_hbm,
                    pa0, pb0, pa1, pb1, wa0, wb0, wa1, wb1,
                    ra0, rb0, ra1, rb1, sem):
        wid = lax.axis_index("s") * NC + lax.axis_index("c")
        tbase = wid * TOK_W
        PA, PB = [pa0, pa1], [pb0, pb1]
        WA, WB = [wa0, wa1], [wb0, wb1]
        RA, RB = [ra0, ra1], [rb0, rb1]
        ds = []
        for h in range(2):
            tb = tbase + h * _HTOK
            pltpu.sync_copy(pos_hbm.at[pl.ds(tb, _HTOK)], PA[h])
            pltpu.sync_copy(pos_hbm.at[pl.ds(S + tb, _HTOK)], PB[h])
            pltpu.sync_copy(wf_hbm.at[pl.ds(tb * 16, _HTOK * 16)], WA[h])
            pltpu.sync_copy(wf_hbm.at[pl.ds((S + tb) * 16, _HTOK * 16)], WB[h])
            ds.append(pltpu.async_copy(os_hbm.at[PA[h]], RA[h], sem))
            ds.append(pltpu.async_copy(os_hbm.at[PB[h]], RB[h], sem))
        for h in range(2):
            tb = tbase + h * _HTOK
            ds[2 * h].wait()
            ds[2 * h + 1].wait()
            acc_v, row_v = RA[h], RB[h]
            for r in range(_HTOK):
                wa = WA[h][pl.ds(r * 16, 16)]
                wb = WB[h][pl.ds(r * 16, 16)]

                def chunk(c, carry, r=r, wa=wa, wb=wb, acc_v=acc_v, row_v=row_v):
                    for u in range(4):
                        sl = pl.ds(c * 64 + u * 16, 16)
                        acc_v[r, sl] = acc_v[r, sl] * wa + row_v[r, sl] * wb
                    return carry
                lax.fori_loop(0, _CHUNKS // 4, chunk, 0)
            pltpu.sync_copy(acc_v, y_hbm.at[pl.ds(tb, _HTOK)])

    return _dispatch_sc, _combine_sc


def _ffn_body(te_s, ntl_s, x_ref, w1_ref, w3_ref, w2_ref, o_ref):
    @pl.when(pl.program_id(0) < ntl_s[0])
    def _():
        bf16 = jnp.bfloat16
        x = x_ref[...].astype(bf16)
        g = lax.dot_general(x, w1_ref[0].astype(bf16), (((1,), (1,)), ((), ())),
                            preferred_element_type=_f32)
        g = g * jax.nn.sigmoid(g)
        u = lax.dot_general(x, w3_ref[0].astype(bf16), (((1,), (1,)), ((), ())),
                            preferred_element_type=_f32)
        h = (g * u).astype(bf16)
        o_ref[...] = lax.dot_general(h, w2_ref[0].astype(bf16), (((1,), (1,)), ((), ())),
                                     preferred_element_type=_f32)


def _ffn(te, ntl, x_sorted, w1, w3, w2):
    grid_spec = pltpu.PrefetchScalarGridSpec(
        num_scalar_prefetch=2,
        grid=(NT,),
        in_specs=[
            pl.BlockSpec((TILE, DIM),
                         lambda i, te_s, ntl_s: (jnp.minimum(i, ntl_s[0] - 1), 0)),
            pl.BlockSpec((1, HID, DIM),
                         lambda i, te_s, ntl_s: (te_s[jnp.minimum(i, ntl_s[0] - 1)], 0, 0)),
            pl.BlockSpec((1, HID, DIM),
                         lambda i, te_s, ntl_s: (te_s[jnp.minimum(i, ntl_s[0] - 1)], 0, 0)),
            pl.BlockSpec((1, DIM, HID),
                         lambda i, te_s, ntl_s: (te_s[jnp.minimum(i, ntl_s[0] - 1)], 0, 0)),
        ],
        out_specs=pl.BlockSpec((TILE, DIM), lambda i, te_s, ntl_s: (i, 0)),
    )
    return pl.pallas_call(
        _ffn_body,
        grid_spec=grid_spec,
        out_shape=jax.ShapeDtypeStruct((CAP, DIM), _f32),
    )(te, ntl, x_sorted, w1, w3, w2)


@jax.jit
def kernel(x, gate_w, w1, w3, w2):
    dispatch_sc, combine_sc = _sc_kernels()
    b, s, d = x.shape
    tokens = x.reshape(S, DIM)
    pos2, te2, ntl2, wflat2 = _gate_route(tokens, gate_w)
    pos = pos2.reshape(N)
    te = te2.reshape(TILE)
    ntl = ntl2.reshape(1)
    wflat = wflat2.reshape(N * 16)
    x_sorted = dispatch_sc(tokens, pos)
    out_sorted = _ffn(te, ntl, x_sorted, w1, w3, w2)
    y = combine_sc(out_sorted, pos, wflat)
    return y.reshape(b, s, d)


# speedup vs baseline: 27.2295x; 1.0273x over previous
import functools

import jax
import jax.numpy as jnp
from jax import lax
from jax.experimental import pallas as pl
from jax.experimental.pallas import tpu as pltpu
from jax.experimental.pallas import tpu_sc as plsc

DIM = 768
HID = 1024
E = 64
K = 2
S = 2048
N = S * K
TILE = 128
NT = 96
CAP = NT * TILE

NC = 2
NS = 16
NW = NC * NS
ROWS_W = N // NW
TOK_W = S // NW

_f32 = jnp.float32
_i32 = jnp.int32


_RB = 1024


def _gate_route_body(tok_ref, gw_ref, pos_ref, te_ref, ntl_ref, wflat_ref,
                     oh_ref, rank_ref):
    x = tok_ref[...]
    gw = gw_ref[...]
    logits = lax.dot_general(x, gw, (((1,), (1,)), ((), ())),
                             preferred_element_type=_f32)

    iota_e = lax.broadcasted_iota(_i32, (S, E), 1)
    m1 = jnp.max(logits, axis=1, keepdims=True)
    i1 = jnp.min(jnp.where(logits == m1, iota_e, E), axis=1, keepdims=True)
    masked = jnp.where(iota_e == i1, _f32(-1e30), logits)
    m2 = jnp.max(masked, axis=1, keepdims=True)
    i2 = jnp.min(jnp.where(masked == m2, iota_e, E), axis=1, keepdims=True)

    e2 = jnp.exp(m2 - m1)
    den = 1.0 + e2
    wflat_ref[pl.ds(0, S), :] = jnp.broadcast_to(1.0 / den, (S, 16))
    wflat_ref[pl.ds(S, S), :] = jnp.broadcast_to(e2 / den, (S, 16))

    iota_se = lax.broadcasted_iota(_i32, (S, E), 1)
    oh_ref[pl.ds(0, S), :] = (i1 == iota_se).astype(_f32)
    oh_ref[pl.ds(S, S), :] = (i2 == iota_se).astype(_f32)

    nblk = N // _RB
    r0 = lax.broadcasted_iota(_i32, (_RB, _RB), 0)
    c0 = lax.broadcasted_iota(_i32, (_RB, _RB), 1)
    tri = (r0 >= c0).astype(jnp.bfloat16)

    def blk1(b, counts):
        oh = oh_ref[pl.ds(b * _RB, _RB), :]
        cum = lax.dot_general(tri, oh.astype(jnp.bfloat16),
                              (((1,), (0,)), ((), ())),
                              preferred_element_type=_f32)
        rank = jnp.sum((cum + counts) * oh, axis=1, keepdims=True) - 1.0
        rank_ref[pl.ds(b * _RB, _RB), :] = rank
        return counts + jnp.sum(oh, axis=0, keepdims=True)

    counts = lax.fori_loop(0, nblk, blk1, jnp.zeros((1, E), _f32))

    ntile = jnp.floor((counts + _f32(TILE - 1)) * _f32(1.0 / TILE))
    e_r = lax.broadcasted_iota(_i32, (E, E), 0)
    e_c = lax.broadcasted_iota(_i32, (E, E), 1)
    excl = (e_r < e_c).astype(_f32)
    off_t = lax.dot_general(ntile, excl, (((1,), (0,)), ((), ())),
                            preferred_element_type=_f32)
    off_r = off_t * _f32(TILE)
    ntl_ref[...] = jnp.sum(ntile, axis=1, keepdims=True).astype(_i32)

    offsel = jnp.sum(oh_ref[...] * off_r, axis=1, keepdims=True)
    pos_ref[...] = (rank_ref[...] + offsel).astype(_i32)

    t_iota = lax.broadcasted_iota(_i32, (TILE, E), 0).astype(_f32)
    te = jnp.sum((off_t <= t_iota).astype(_f32), axis=1, keepdims=True) - 1.0
    te_ref[...] = te.astype(_i32)


def _gate_route(tokens, gate_w):
    return pl.pallas_call(
        _gate_route_body,
        out_shape=(
            jax.ShapeDtypeStruct((N, 1), _i32),
            jax.ShapeDtypeStruct((TILE, 1), _i32),
            jax.ShapeDtypeStruct((1, 1), _i32),
            jax.ShapeDtypeStruct((N, 16), _f32),
        ),
        scratch_shapes=[
            pltpu.VMEM((N, E), _f32),
            pltpu.VMEM((N, 1), _f32),
        ],
    )(tokens, gate_w)


_HTOK = TOK_W // 2
_CHUNKS = DIM // 16


@functools.lru_cache(maxsize=1)
def _sc_kernels():
    mesh = plsc.VectorSubcoreMesh(core_axis_name="c", subcore_axis_name="s")

    @functools.partial(
        pl.kernel,
        out_type=jax.ShapeDtypeStruct((CAP, DIM), _f32),
        mesh=mesh,
        scratch_types=[
            pltpu.VMEM((64,), _i32),
            pltpu.VMEM((64,), _i32),
            pltpu.VMEM((64,), _i32),
            pltpu.VMEM((64,), _i32),
            pltpu.VMEM((64, DIM), _f32),
            pltpu.VMEM((64, DIM), _f32),
            pltpu.SemaphoreType.DMA,
            pltpu.SemaphoreType.DMA,
        ],
    )
    def _dispatch_sc(tok_hbm, pos_hbm, xs_hbm,
                     src0, src1, pos0, pos1, rows0, rows1, gsem, ssem):
        wid = lax.axis_index("s") * NC + lax.axis_index("c")
        base = wid * ROWS_W
        srcs, poss, rows = [src0, src1], [pos0, pos1], [rows0, rows1]
        gd = []
        for h in range(2):
            b = base + h * 64
            for c in range(4):
                srcs[h][pl.ds(c * 16, 16)] = (
                    lax.iota(_i32, 16) + (b + c * 16)) & (S - 1)
            pltpu.sync_copy(pos_hbm.at[pl.ds(b, 64)], poss[h])
            gd.append(pltpu.async_copy(tok_hbm.at[srcs[h]], rows[h], gsem))
        sd = []
        for h in range(2):
            gd[h].wait()
            sd.append(pltpu.async_copy(rows[h], xs_hbm.at[poss[h]], ssem))
        for d in sd:
            d.wait()

    @functools.partial(
        pl.kernel,
        out_type=jax.ShapeDtypeStruct((S, DIM), _f32),
        mesh=mesh,
        scratch_types=[
            pltpu.VMEM((_HTOK,), _i32),
            pltpu.VMEM((_HTOK,), _i32),
            pltpu.VMEM((_HTOK,), _i32),
            pltpu.VMEM((_HTOK,), _i32),
            pltpu.VMEM((_HTOK * 16,), _f32),
            pltpu.VMEM((_HTOK * 16,), _f32),
            pltpu.VMEM((_HTOK * 16,), _f32),
            pltpu.VMEM((_HTOK * 16,), _f32),
            pltpu.VMEM((_HTOK, DIM), _f32),
            pltpu.VMEM((_HTOK, DIM), _f32),
            pltpu.VMEM((_HTOK, DIM), _f32),
            pltpu.VMEM((_HTOK, DIM), _f32),
            pltpu.SemaphoreType.DMA,
        ],
    )
    def _combine_sc(os_hbm, pos_hbm, wf_hbm, y_hbm,
                    pa0, pb0, pa1, pb1, wa0, wb0, wa1, wb1,
                    ra0, rb0, ra1, rb1, sem):
        wid = lax.axis_index("s") * NC + lax.axis_index("c")
        tbase = wid * TOK_W
        PA, PB = [pa0, pa1], [pb0, pb1]
        WA, WB = [wa0, wa1], [wb0, wb1]
        RA, RB = [ra0, ra1], [rb0, rb1]
        ds = []
        for h in range(2):
            tb = tbase + h * _HTOK
            pltpu.sync_copy(pos_hbm.at[pl.ds(tb, _HTOK)], PA[h])
            pltpu.sync_copy(pos_hbm.at[pl.ds(S + tb, _HTOK)], PB[h])
            pltpu.sync_copy(wf_hbm.at[pl.ds(tb * 16, _HTOK * 16)], WA[h])
            pltpu.sync_copy(wf_hbm.at[pl.ds((S + tb) * 16, _HTOK * 16)], WB[h])
            ds.append(pltpu.async_copy(os_hbm.at[PA[h]], RA[h], sem))
            ds.append(pltpu.async_copy(os_hbm.at[PB[h]], RB[h], sem))
        for h in range(2):
            tb = tbase + h * _HTOK
            ds[2 * h].wait()
            ds[2 * h + 1].wait()
            acc_v, row_v, wav, wbv = RA[h], RB[h], WA[h], WB[h]

            def row_body(r, carry, acc_v=acc_v, row_v=row_v, wav=wav, wbv=wbv):
                wa = wav[pl.ds(r * 16, 16)]
                wb = wbv[pl.ds(r * 16, 16)]
                for c in range(_CHUNKS):
                    sl = pl.ds(c * 16, 16)
                    acc_v[r, sl] = acc_v[r, sl] * wa + row_v[r, sl] * wb
                return carry
            lax.fori_loop(0, _HTOK, row_body, 0)
            pltpu.sync_copy(acc_v, y_hbm.at[pl.ds(tb, _HTOK)])

    return _dispatch_sc, _combine_sc


def _ffn_body(te_s, ntl_s, x_ref, w1_ref, w3_ref, w2_ref, o_ref):
    @pl.when(pl.program_id(0) < ntl_s[0])
    def _():
        bf16 = jnp.bfloat16
        x = x_ref[...].astype(bf16)
        g = lax.dot_general(x, w1_ref[0].astype(bf16), (((1,), (1,)), ((), ())),
                            preferred_element_type=_f32)
        g = g * jax.nn.sigmoid(g)
        u = lax.dot_general(x, w3_ref[0].astype(bf16), (((1,), (1,)), ((), ())),
                            preferred_element_type=_f32)
        h = (g * u).astype(bf16)
        o_ref[...] = lax.dot_general(h, w2_ref[0].astype(bf16), (((1,), (1,)), ((), ())),
                                     preferred_element_type=_f32)


def _ffn(te, ntl, x_sorted, w1, w3, w2):
    grid_spec = pltpu.PrefetchScalarGridSpec(
        num_scalar_prefetch=2,
        grid=(NT,),
        in_specs=[
            pl.BlockSpec((TILE, DIM),
                         lambda i, te_s, ntl_s: (jnp.minimum(i, ntl_s[0] - 1), 0)),
            pl.BlockSpec((1, HID, DIM),
                         lambda i, te_s, ntl_s: (te_s[jnp.minimum(i, ntl_s[0] - 1)], 0, 0)),
            pl.BlockSpec((1, HID, DIM),
                         lambda i, te_s, ntl_s: (te_s[jnp.minimum(i, ntl_s[0] - 1)], 0, 0)),
            pl.BlockSpec((1, DIM, HID),
                         lambda i, te_s, ntl_s: (te_s[jnp.minimum(i, ntl_s[0] - 1)], 0, 0)),
        ],
        out_specs=pl.BlockSpec((TILE, DIM), lambda i, te_s, ntl_s: (i, 0)),
    )
    return pl.pallas_call(
        _ffn_body,
        grid_spec=grid_spec,
        out_shape=jax.ShapeDtypeStruct((CAP, DIM), _f32),
    )(te, ntl, x_sorted, w1, w3, w2)


@jax.jit
def kernel(x, gate_w, w1, w3, w2):
    dispatch_sc, combine_sc = _sc_kernels()
    b, s, d = x.shape
    tokens = x.reshape(S, DIM)
    pos2, te2, ntl2, wflat2 = _gate_route(tokens, gate_w)
    pos = pos2.reshape(N)
    te = te2.reshape(TILE)
    ntl = ntl2.reshape(1)
    wflat = wflat2.reshape(N * 16)
    x_sorted = dispatch_sc(tokens, pos)
    out_sorted = _ffn(te, ntl, x_sorted, w1, w3, w2)
    y = combine_sc(out_sorted, pos, wflat)
    return y.reshape(b, s, d)


# speedup vs baseline: 28.2608x; 1.0379x over previous
import functools

import jax
import jax.numpy as jnp
from jax import lax
from jax.experimental import pallas as pl
from jax.experimental.pallas import tpu as pltpu
from jax.experimental.pallas import tpu_sc as plsc

DIM = 768
HID = 1024
E = 64
K = 2
S = 2048
N = S * K
TILE = 128
NT = 96
CAP = NT * TILE

NC = 2
NS = 16
NW = NC * NS
ROWS_W = N // NW
TOK_W = S // NW

_f32 = jnp.float32
_i32 = jnp.int32


_RB = 1024


def _gate_route_body(tok_ref, gw_ref, pos_ref, te_ref, ntl_ref, wflat_ref,
                     oh_ref, rank_ref):
    x = tok_ref[...]
    gw = gw_ref[...]
    logits = lax.dot_general(x, gw, (((1,), (1,)), ((), ())),
                             preferred_element_type=_f32)

    iota_e = lax.broadcasted_iota(_i32, (S, E), 1)
    m1 = jnp.max(logits, axis=1, keepdims=True)
    i1 = jnp.min(jnp.where(logits == m1, iota_e, E), axis=1, keepdims=True)
    masked = jnp.where(iota_e == i1, _f32(-1e30), logits)
    m2 = jnp.max(masked, axis=1, keepdims=True)
    i2 = jnp.min(jnp.where(masked == m2, iota_e, E), axis=1, keepdims=True)

    e2 = jnp.exp(m2 - m1)
    den = 1.0 + e2
    wflat_ref[pl.ds(0, S), :] = jnp.broadcast_to(1.0 / den, (S, 16))
    wflat_ref[pl.ds(S, S), :] = jnp.broadcast_to(e2 / den, (S, 16))

    iota_se = lax.broadcasted_iota(_i32, (S, E), 1)
    oh_ref[pl.ds(0, S), :] = (i1 == iota_se).astype(_f32)
    oh_ref[pl.ds(S, S), :] = (i2 == iota_se).astype(_f32)

    nblk = N // _RB
    r0 = lax.broadcasted_iota(_i32, (_RB, _RB), 0)
    c0 = lax.broadcasted_iota(_i32, (_RB, _RB), 1)
    tri = (r0 >= c0).astype(jnp.bfloat16)

    def blk1(b, counts):
        oh = oh_ref[pl.ds(b * _RB, _RB), :]
        cum = lax.dot_general(tri, oh.astype(jnp.bfloat16),
                              (((1,), (0,)), ((), ())),
                              preferred_element_type=_f32)
        rank = jnp.sum((cum + counts) * oh, axis=1, keepdims=True) - 1.0
        rank_ref[pl.ds(b * _RB, _RB), :] = rank
        return counts + jnp.sum(oh, axis=0, keepdims=True)

    counts = lax.fori_loop(0, nblk, blk1, jnp.zeros((1, E), _f32))

    ntile = jnp.floor((counts + _f32(TILE - 1)) * _f32(1.0 / TILE))
    e_r = lax.broadcasted_iota(_i32, (E, E), 0)
    e_c = lax.broadcasted_iota(_i32, (E, E), 1)
    excl = (e_r < e_c).astype(_f32)
    off_t = lax.dot_general(ntile, excl, (((1,), (0,)), ((), ())),
                            preferred_element_type=_f32)
    off_r = off_t * _f32(TILE)
    ntl_ref[...] = jnp.sum(ntile, axis=1, keepdims=True).astype(_i32)

    offsel = jnp.sum(oh_ref[...] * off_r, axis=1, keepdims=True)
    pos_ref[...] = (rank_ref[...] + offsel).astype(_i32)

    t_iota = lax.broadcasted_iota(_i32, (TILE, E), 0).astype(_f32)
    te = jnp.sum((off_t <= t_iota).astype(_f32), axis=1, keepdims=True) - 1.0
    te_ref[...] = te.astype(_i32)


def _gate_route(tokens, gate_w):
    return pl.pallas_call(
        _gate_route_body,
        out_shape=(
            jax.ShapeDtypeStruct((N, 1), _i32),
            jax.ShapeDtypeStruct((TILE, 1), _i32),
            jax.ShapeDtypeStruct((1, 1), _i32),
            jax.ShapeDtypeStruct((N, 16), _f32),
        ),
        scratch_shapes=[
            pltpu.VMEM((N, E), _f32),
            pltpu.VMEM((N, 1), _f32),
        ],
    )(tokens, gate_w)


_HTOK = TOK_W // 2
_CHUNKS = DIM // 16


@functools.lru_cache(maxsize=1)
def _sc_kernels():
    mesh = plsc.VectorSubcoreMesh(core_axis_name="c", subcore_axis_name="s")

    @functools.partial(
        pl.kernel,
        out_type=jax.ShapeDtypeStruct((CAP, DIM), _f32),
        mesh=mesh,
        scratch_types=[
            pltpu.VMEM((64,), _i32),
            pltpu.VMEM((64,), _i32),
            pltpu.VMEM((64,), _i32),
            pltpu.VMEM((64,), _i32),
            pltpu.VMEM((64, DIM), _f32),
            pltpu.VMEM((64, DIM), _f32),
            pltpu.SemaphoreType.DMA,
            pltpu.SemaphoreType.DMA,
        ],
    )
    def _dispatch_sc(tok_hbm, pos_hbm, xs_hbm,
                     src0, src1, pos0, pos1, rows0, rows1, gsem, ssem):
        wid = lax.axis_index("s") * NC + lax.axis_index("c")
        base = wid * ROWS_W
        srcs, poss, rows = [src0, src1], [pos0, pos1], [rows0, rows1]
        gd = []
        for h in range(2):
            b = base + h * 64
            for c in range(4):
                srcs[h][pl.ds(c * 16, 16)] = (
                    lax.iota(_i32, 16) + (b + c * 16)) & (S - 1)
            pltpu.sync_copy(pos_hbm.at[pl.ds(b, 64)], poss[h])
            gd.append(pltpu.async_copy(tok_hbm.at[srcs[h]], rows[h], gsem))
        sd = []
        for h in range(2):
            gd[h].wait()
            sd.append(pltpu.async_copy(rows[h], xs_hbm.at[poss[h]], ssem))
        for d in sd:
            d.wait()

    @functools.partial(
        pl.kernel,
        out_type=jax.ShapeDtypeStruct((S, DIM), _f32),
        mesh=mesh,
        scratch_types=[
            pltpu.VMEM((_HTOK,), _i32),
            pltpu.VMEM((_HTOK,), _i32),
            pltpu.VMEM((_HTOK,), _i32),
            pltpu.VMEM((_HTOK,), _i32),
            pltpu.VMEM((_HTOK * 16,), _f32),
            pltpu.VMEM((_HTOK * 16,), _f32),
            pltpu.VMEM((_HTOK * 16,), _f32),
            pltpu.VMEM((_HTOK * 16,), _f32),
            pltpu.VMEM((_HTOK, DIM), _f32),
            pltpu.VMEM((_HTOK, DIM), _f32),
            pltpu.VMEM((_HTOK, DIM), _f32),
            pltpu.VMEM((_HTOK, DIM), _f32),
            pltpu.SemaphoreType.DMA,
        ],
    )
    def _combine_sc(os_hbm, pos_hbm, wf_hbm, y_hbm,
                    pa0, pb0, pa1, pb1, wa0, wb0, wa1, wb1,
                    ra0, rb0, ra1, rb1, sem):
        wid = lax.axis_index("s") * NC + lax.axis_index("c")
        tbase = wid * TOK_W
        PA, PB = [pa0, pa1], [pb0, pb1]
        WA, WB = [wa0, wa1], [wb0, wb1]
        RA, RB = [ra0, ra1], [rb0, rb1]
        ds = []
        for h in range(2):
            tb = tbase + h * _HTOK
            pltpu.sync_copy(pos_hbm.at[pl.ds(tb, _HTOK)], PA[h])
            pltpu.sync_copy(pos_hbm.at[pl.ds(S + tb, _HTOK)], PB[h])
            pltpu.sync_copy(wf_hbm.at[pl.ds(tb * 16, _HTOK * 16)], WA[h])
            pltpu.sync_copy(wf_hbm.at[pl.ds((S + tb) * 16, _HTOK * 16)], WB[h])
            ds.append(pltpu.async_copy(os_hbm.at[PA[h]], RA[h], sem))
            ds.append(pltpu.async_copy(os_hbm.at[PB[h]], RB[h], sem))
        for h in range(2):
            tb = tbase + h * _HTOK
            ds[2 * h].wait()
            ds[2 * h + 1].wait()
            acc_v, row_v, wav, wbv = RA[h], RB[h], WA[h], WB[h]

            def row_body(r, carry, acc_v=acc_v, row_v=row_v, wav=wav, wbv=wbv):
                wa = wav[pl.ds(r * 16, 16)]
                wb = wbv[pl.ds(r * 16, 16)]
                for c in range(_CHUNKS):
                    sl = pl.ds(c * 16, 16)
                    acc_v[r, sl] = acc_v[r, sl] * wa + row_v[r, sl] * wb
                return carry
            lax.fori_loop(0, _HTOK, row_body, 0)
            pltpu.sync_copy(acc_v, y_hbm.at[pl.ds(tb, _HTOK)])

    return _dispatch_sc, _combine_sc


def _ffn_body(te_s, ntl_s, x_ref, w1_ref, w3_ref, w2_ref, o_ref):
    @pl.when(pl.program_id(0) < ntl_s[0])
    def _():
        bf16 = jnp.bfloat16
        x = x_ref[...].astype(bf16)
        g = lax.dot_general(x, w1_ref[0].astype(bf16), (((1,), (1,)), ((), ())),
                            preferred_element_type=_f32)
        g = g * jax.nn.sigmoid(g)
        u = lax.dot_general(x, w3_ref[0].astype(bf16), (((1,), (1,)), ((), ())),
                            preferred_element_type=_f32)
        h = (g * u).astype(bf16)
        o_ref[...] = lax.dot_general(h, w2_ref[0].astype(bf16), (((1,), (1,)), ((), ())),
                                     preferred_element_type=_f32)


def _ffn(te, ntl, x_sorted, w1, w3, w2):
    grid_spec = pltpu.PrefetchScalarGridSpec(
        num_scalar_prefetch=2,
        grid=(NT,),
        in_specs=[
            pl.BlockSpec((TILE, DIM),
                         lambda i, te_s, ntl_s: (jnp.minimum(i, ntl_s[0] - 1), 0)),
            pl.BlockSpec((1, HID, DIM),
                         lambda i, te_s, ntl_s: (te_s[jnp.minimum(i, ntl_s[0] - 1)], 0, 0)),
            pl.BlockSpec((1, HID, DIM),
                         lambda i, te_s, ntl_s: (te_s[jnp.minimum(i, ntl_s[0] - 1)], 0, 0)),
            pl.BlockSpec((1, DIM, HID),
                         lambda i, te_s, ntl_s: (te_s[jnp.minimum(i, ntl_s[0] - 1)], 0, 0)),
        ],
        out_specs=pl.BlockSpec((TILE, DIM),
                               lambda i, te_s, ntl_s: (jnp.minimum(i, ntl_s[0] - 1), 0)),
    )
    return pl.pallas_call(
        _ffn_body,
        grid_spec=grid_spec,
        out_shape=jax.ShapeDtypeStruct((CAP, DIM), _f32),
    )(te, ntl, x_sorted, w1, w3, w2)


@jax.jit
def kernel(x, gate_w, w1, w3, w2):
    dispatch_sc, combine_sc = _sc_kernels()
    b, s, d = x.shape
    tokens = x.reshape(S, DIM)
    pos2, te2, ntl2, wflat2 = _gate_route(tokens, gate_w)
    pos = pos2.reshape(N)
    te = te2.reshape(TILE)
    ntl = ntl2.reshape(1)
    wflat = wflat2.reshape(N * 16)
    x_sorted = dispatch_sc(tokens, pos)
    out_sorted = _ffn(te, ntl, x_sorted, w1, w3, w2)
    y = combine_sc(out_sorted, pos, wflat)
    return y.reshape(b, s, d)
